# baseline (speedup 1.0000x reference)
import functools

import jax
import jax.numpy as jnp
from jax import lax
from jax.experimental import pallas as pl
from jax.experimental.pallas import tpu as pltpu
from jax.experimental.pallas import tpu_sc as plsc

B = 16384
H = 50
E = 32
V = 1000000

NC = 2
NS = 16
NW = NC * NS
ROWS_PER_W = B // NW
CB = 32
NCHUNK = ROWS_PER_W // CB
CHUNK_I = CB * H

CW = 512
NFULL = V // CW
TAIL = V - NFULL * CW
MAXI = (NFULL + 1 + NW - 1) // NW


def _sc_repack(tT, tail_rows):
    mesh = plsc.VectorSubcoreMesh(core_axis_name="c", subcore_axis_name="s")

    CWP = CW + 1

    @functools.partial(
        pl.kernel,
        out_type=jax.ShapeDtypeStruct((V * E,), jnp.float32),
        mesh=mesh,
        compiler_params=pltpu.CompilerParams(needs_layout_passes=False),
        scratch_types=[
            pltpu.VMEM((E, CWP), jnp.float32),
            pltpu.VMEM((E, CWP), jnp.float32),
            pltpu.VMEM((CW * E,), jnp.float32),
            pltpu.VMEM((TAIL, E), jnp.float32),
            pltpu.SemaphoreType.DMA,
            pltpu.SemaphoreType.DMA,
        ],
    )
    def body(tT_hbm, tail_hbm, out_hbm, buf0, buf1, obuf, tailbuf,
             sem0, sem1):
        wid = lax.axis_index("s") * NC + lax.axis_index("c")
        iota = lax.iota(jnp.int32, 16)
        row_lo = iota
        row_hi = iota + 16
        bufs = (buf0, buf1)
        sems = (sem0, sem1)

        def start_fetch(bi, k):
            @pl.when(k < NFULL)
            def _():
                pltpu.async_copy(tT_hbm.at[:, pl.ds(k * CW, CW)],
                                 bufs[bi].at[:, pl.ds(0, CW)], sems[bi])

        def wait_fetch(bi, k):
            @pl.when(k < NFULL)
            def _():
                pltpu.make_async_copy(tT_hbm.at[:, pl.ds(0, CW)],
                                      bufs[bi].at[:, pl.ds(0, CW)],
                                      sems[bi]).wait()

        def transpose_write(bi, k):
            @pl.when(k < NFULL)
            def _():
                buf = bufs[bi]

                def rr_body(rr, _):
                    col = jnp.full((16,), rr, jnp.int32)
                    obuf[pl.ds(rr * E, 16)] = plsc.load_gather(
                        buf, [row_lo, col])
                    obuf[pl.ds(rr * E + 16, 16)] = plsc.load_gather(
                        buf, [row_hi, col])
                    return 0

                lax.fori_loop(0, CW, rr_body, 0, unroll=8)
                pltpu.sync_copy(obuf, out_hbm.at[pl.ds(k * CW * E, CW * E)])

        start_fetch(0, wid)
        def pair_body(t, _):
            k0 = wid + (2 * NW) * t
            wait_fetch(0, k0)
            start_fetch(1, k0 + NW)
            transpose_write(0, k0)
            wait_fetch(1, k0 + NW)
            start_fetch(0, k0 + 2 * NW)
            transpose_write(1, k0 + NW)
            return 0
        lax.fori_loop(0, MAXI // 2, pair_body, 0)

        @pl.when(wid == NFULL % NW)
        def _():
            col0 = NFULL * CW
            pltpu.sync_copy(tail_hbm, tailbuf)
            for r in range(TAIL):
                obuf[pl.ds(r * E, 16)] = tailbuf[r, 0:16]
                obuf[pl.ds(r * E + 16, 16)] = tailbuf[r, 16:32]
            pltpu.sync_copy(obuf.at[pl.ds(0, TAIL * E)],
                            out_hbm.at[pl.ds(col0 * E, TAIL * E)])

    return body(tT, tail_rows)


def _sc_pooled_sum(xflat, table):
    mesh = plsc.VectorSubcoreMesh(core_axis_name="c", subcore_axis_name="s")

    @functools.partial(
        pl.kernel,
        out_type=jax.ShapeDtypeStruct((B * E,), jnp.float32),
        mesh=mesh,
        compiler_params=pltpu.CompilerParams(use_tc_tiling_on_sc=False),
        scratch_types=[
            pltpu.VMEM((CHUNK_I,), jnp.int32),
            pltpu.VMEM((CHUNK_I,), jnp.int32),
            pltpu.VMEM((CHUNK_I, E), jnp.float32),
            pltpu.VMEM((CHUNK_I, E), jnp.float32),
            pltpu.VMEM((CB * E,), jnp.float32),
            pltpu.SemaphoreType.DMA,
            pltpu.SemaphoreType.DMA,
        ],
    )
    def body(x_hbm, table_hbm, out_hbm, idx0, idx1, rows0, rows1, stage,
             sem0, sem1):
        wid = lax.axis_index("s") * NC + lax.axis_index("c")
        ibase = wid * (ROWS_PER_W * H)
        obase = wid * (ROWS_PER_W * E)

        idx = (idx0, idx1)
        rows = (rows0, rows1)
        sems = (sem0, sem1)
        handles = [None, None]

        pltpu.sync_copy(x_hbm.at[pl.ds(ibase, CHUNK_I)], idx[0])
        handles[0] = pltpu.async_copy(table_hbm.at[idx[0]], rows[0], sems[0])

        for c in range(NCHUNK):
            cur = c % 2
            nxt = (c + 1) % 2
            if c + 1 < NCHUNK:
                pltpu.sync_copy(
                    x_hbm.at[pl.ds(ibase + (c + 1) * CHUNK_I, CHUNK_I)],
                    idx[nxt])
                handles[nxt] = pltpu.async_copy(
                    table_hbm.at[idx[nxt]], rows[nxt], sems[nxt])
            handles[cur].wait()
            rref = rows[cur]

            def row_body(bi, _, rref=rref):
                base = bi * H
                a0 = rref[base, 0:16]
                a1 = rref[base, 16:32]
                for j in range(1, H):
                    a0 = a0 + rref[base + j, 0:16]
                    a1 = a1 + rref[base + j, 16:32]
                stage[pl.ds(bi * E, 16)] = a0
                stage[pl.ds(bi * E + 16, 16)] = a1
                return 0

            lax.fori_loop(0, CB, row_body, 0)
            pltpu.sync_copy(
                stage, out_hbm.at[pl.ds(obase + c * (CB * E), CB * E)])

    return body(xflat, table)


def _tc_tail(pooled_sum, W, b2):
    BB = 2048

    def body(p_ref, w_ref, b_ref, o_ref):
        p = p_ref[...] * (1.0 / H)
        h = jnp.dot(p, w_ref[...], preferred_element_type=jnp.float32)
        h = jnp.maximum(h + b_ref[...], 0.0)
        m = jnp.max(h, axis=1, keepdims=True)
        e = jnp.exp(h - m)
        o_ref[...] = (h - m) - jnp.log(jnp.sum(e, axis=1, keepdims=True))

    return pl.pallas_call(
        body,
        grid=(B // BB,),
        in_specs=[
            pl.BlockSpec((BB, E), lambda i: (i, 0)),
            pl.BlockSpec((E, 2), lambda i: (0, 0)),
            pl.BlockSpec((1, 2), lambda i: (0, 0)),
        ],
        out_specs=pl.BlockSpec((BB, 2), lambda i: (i, 0)),
        out_shape=jax.ShapeDtypeStruct((B, 2), jnp.float32),
    )(pooled_sum, W, b2)


def kernel(x, table, W, b):
    xflat = x.astype(jnp.int32).reshape(B * H)
    packed = _sc_repack(table.T, table[NFULL * CW:])
    table_lin = packed.reshape(V, E)
    pooled_sum = _sc_pooled_sum(xflat, table_lin).reshape(B, E)
    return _tc_tail(pooled_sum, W, b.reshape(1, 2))


# speedup vs baseline: 1.5525x; 1.5525x over previous
import functools

import jax
import jax.numpy as jnp
from jax import lax
from jax.experimental import pallas as pl
from jax.experimental.pallas import tpu as pltpu
from jax.experimental.pallas import tpu_sc as plsc

B = 16384
H = 50
E = 32
V = 1000000

NC = 2
NS = 16
NW = NC * NS
ROWS_PER_W = B // NW
CB = 32
NCHUNK = ROWS_PER_W // CB
CHUNK_I = CB * H

CW = 512
NFULL = V // CW
TAIL = V - NFULL * CW
MAXI = (NFULL + 1 + NW - 1) // NW


def _tc_repack(tT):
    BLK = 2048
    R = E * 4
    OB = BLK // 4

    def body(t_ref, o_ref):
        w = jnp.transpose(t_ref[...])
        w3 = w.reshape(OB, 4, E)
        o_ref[...] = jnp.concatenate(
            [w3[:, j, :] for j in range(4)], axis=1)

    return pl.pallas_call(
        body,
        grid=(pl.cdiv(V, BLK),),
        in_specs=[pl.BlockSpec((E, BLK), lambda i: (0, i))],
        out_specs=pl.BlockSpec((OB, R), lambda i: (i, 0)),
        out_shape=jax.ShapeDtypeStruct((V * E // R, R), jnp.float32),
    )(tT)


def _sc_repack(tT, tail_rows):
    mesh = plsc.VectorSubcoreMesh(core_axis_name="c", subcore_axis_name="s")

    CWP = CW + 1

    @functools.partial(
        pl.kernel,
        out_type=jax.ShapeDtypeStruct((V * E,), jnp.float32),
        mesh=mesh,
        compiler_params=pltpu.CompilerParams(needs_layout_passes=False),
        scratch_types=[
            pltpu.VMEM((E, CWP), jnp.float32),
            pltpu.VMEM((E, CWP), jnp.float32),
            pltpu.VMEM((CW * E,), jnp.float32),
            pltpu.VMEM((TAIL, E), jnp.float32),
            pltpu.SemaphoreType.DMA,
            pltpu.SemaphoreType.DMA,
        ],
    )
    def body(tT_hbm, tail_hbm, out_hbm, buf0, buf1, obuf, tailbuf,
             sem0, sem1):
        wid = lax.axis_index("s") * NC + lax.axis_index("c")
        iota = lax.iota(jnp.int32, 16)
        row_lo = iota
        row_hi = iota + 16
        bufs = (buf0, buf1)
        sems = (sem0, sem1)

        def start_fetch(bi, k):
            @pl.when(k < NFULL)
            def _():
                pltpu.async_copy(tT_hbm.at[:, pl.ds(k * CW, CW)],
                                 bufs[bi].at[:, pl.ds(0, CW)], sems[bi])

        def wait_fetch(bi, k):
            @pl.when(k < NFULL)
            def _():
                pltpu.make_async_copy(tT_hbm.at[:, pl.ds(0, CW)],
                                      bufs[bi].at[:, pl.ds(0, CW)],
                                      sems[bi]).wait()

        def transpose_write(bi, k):
            @pl.when(k < NFULL)
            def _():
                buf = bufs[bi]

                def rr_body(rr, _):
                    col = jnp.full((16,), rr, jnp.int32)
                    obuf[pl.ds(rr * E, 16)] = plsc.load_gather(
                        buf, [row_lo, col])
                    obuf[pl.ds(rr * E + 16, 16)] = plsc.load_gather(
                        buf, [row_hi, col])
                    return 0

                lax.fori_loop(0, CW, rr_body, 0, unroll=8)
                pltpu.sync_copy(obuf, out_hbm.at[pl.ds(k * CW * E, CW * E)])

        start_fetch(0, wid)
        def pair_body(t, _):
            k0 = wid + (2 * NW) * t
            wait_fetch(0, k0)
            start_fetch(1, k0 + NW)
            transpose_write(0, k0)
            wait_fetch(1, k0 + NW)
            start_fetch(0, k0 + 2 * NW)
            transpose_write(1, k0 + NW)
            return 0
        lax.fori_loop(0, MAXI // 2, pair_body, 0)

        @pl.when(wid == NFULL % NW)
        def _():
            col0 = NFULL * CW
            pltpu.sync_copy(tail_hbm, tailbuf)
            for r in range(TAIL):
                obuf[pl.ds(r * E, 16)] = tailbuf[r, 0:16]
                obuf[pl.ds(r * E + 16, 16)] = tailbuf[r, 16:32]
            pltpu.sync_copy(obuf.at[pl.ds(0, TAIL * E)],
                            out_hbm.at[pl.ds(col0 * E, TAIL * E)])

    return body(tT, tail_rows)


def _sc_pooled_sum(xflat, table):
    mesh = plsc.VectorSubcoreMesh(core_axis_name="c", subcore_axis_name="s")

    @functools.partial(
        pl.kernel,
        out_type=jax.ShapeDtypeStruct((B * E,), jnp.float32),
        mesh=mesh,
        compiler_params=pltpu.CompilerParams(use_tc_tiling_on_sc=False),
        scratch_types=[
            pltpu.VMEM((CHUNK_I,), jnp.int32),
            pltpu.VMEM((CHUNK_I,), jnp.int32),
            pltpu.VMEM((CHUNK_I, E), jnp.float32),
            pltpu.VMEM((CHUNK_I, E), jnp.float32),
            pltpu.VMEM((CB * E,), jnp.float32),
            pltpu.SemaphoreType.DMA,
            pltpu.SemaphoreType.DMA,
        ],
    )
    def body(x_hbm, table_hbm, out_hbm, idx0, idx1, rows0, rows1, stage,
             sem0, sem1):
        wid = lax.axis_index("s") * NC + lax.axis_index("c")
        ibase = wid * (ROWS_PER_W * H)
        obase = wid * (ROWS_PER_W * E)

        idx = (idx0, idx1)
        rows = (rows0, rows1)
        sems = (sem0, sem1)
        handles = [None, None]

        pltpu.sync_copy(x_hbm.at[pl.ds(ibase, CHUNK_I)], idx[0])
        handles[0] = pltpu.async_copy(table_hbm.at[idx[0]], rows[0], sems[0])

        for c in range(NCHUNK):
            cur = c % 2
            nxt = (c + 1) % 2
            if c + 1 < NCHUNK:
                pltpu.sync_copy(
                    x_hbm.at[pl.ds(ibase + (c + 1) * CHUNK_I, CHUNK_I)],
                    idx[nxt])
                handles[nxt] = pltpu.async_copy(
                    table_hbm.at[idx[nxt]], rows[nxt], sems[nxt])
            handles[cur].wait()
            rref = rows[cur]

            def row_body(bi, _, rref=rref):
                base = bi * H
                a0 = rref[base, 0:16]
                a1 = rref[base, 16:32]
                for j in range(1, H):
                    a0 = a0 + rref[base + j, 0:16]
                    a1 = a1 + rref[base + j, 16:32]
                stage[pl.ds(bi * E, 16)] = a0
                stage[pl.ds(bi * E + 16, 16)] = a1
                return 0

            lax.fori_loop(0, CB, row_body, 0)
            pltpu.sync_copy(
                stage, out_hbm.at[pl.ds(obase + c * (CB * E), CB * E)])

    return body(xflat, table)


def _tc_tail(pooled_sum, W, b2):
    BB = 2048

    def body(p_ref, w_ref, b_ref, o_ref):
        p = p_ref[...] * (1.0 / H)
        h = jnp.dot(p, w_ref[...], preferred_element_type=jnp.float32)
        h = jnp.maximum(h + b_ref[...], 0.0)
        m = jnp.max(h, axis=1, keepdims=True)
        e = jnp.exp(h - m)
        o_ref[...] = (h - m) - jnp.log(jnp.sum(e, axis=1, keepdims=True))

    return pl.pallas_call(
        body,
        grid=(B // BB,),
        in_specs=[
            pl.BlockSpec((BB, E), lambda i: (i, 0)),
            pl.BlockSpec((E, 2), lambda i: (0, 0)),
            pl.BlockSpec((1, 2), lambda i: (0, 0)),
        ],
        out_specs=pl.BlockSpec((BB, 2), lambda i: (i, 0)),
        out_shape=jax.ShapeDtypeStruct((B, 2), jnp.float32),
    )(pooled_sum, W, b2)


def kernel(x, table, W, b):
    xflat = x.astype(jnp.int32).reshape(B * H)
    packed = _tc_repack(table.T)
    table_lin = packed.reshape(V, E)
    pooled_sum = _sc_pooled_sum(xflat, table_lin).reshape(B, E)
    return _tc_tail(pooled_sum, W, b.reshape(1, 2))


# speedup vs baseline: 1.7402x; 1.1209x over previous
import functools

import jax
import jax.numpy as jnp
from jax import lax
from jax.experimental import pallas as pl
from jax.experimental.pallas import tpu as pltpu
from jax.experimental.pallas import tpu_sc as plsc

B = 16384
H = 50
E = 32
V = 1000000

NC = 2
NS = 16
NW = NC * NS
ROWS_PER_W = B // NW
CB = 32
NCHUNK = ROWS_PER_W // CB
CHUNK_I = CB * H

CW = 512
NFULL = V // CW
TAIL = V - NFULL * CW
MAXI = (NFULL + 1 + NW - 1) // NW


def _tc_repack(tT):
    BLK = 2048
    R = E * 4
    OB = BLK // 4

    def body(t_ref, o_ref):
        v = t_ref[...]
        o_ref[...] = jnp.concatenate(
            [jnp.transpose(v[:, j * OB:(j + 1) * OB]) for j in range(4)],
            axis=1)

    ngrid = pl.cdiv(V, BLK)
    return pl.pallas_call(
        body,
        grid=(ngrid,),
        in_specs=[pl.BlockSpec((E, BLK), lambda i: (0, i))],
        out_specs=pl.BlockSpec((OB, R), lambda i: (i, 0)),
        out_shape=jax.ShapeDtypeStruct((ngrid * OB, R), jnp.float32),
    )(tT)


def _sc_pooled_sum(xflat, table):
    mesh = plsc.VectorSubcoreMesh(core_axis_name="c", subcore_axis_name="s")

    @functools.partial(
        pl.kernel,
        out_type=jax.ShapeDtypeStruct((B * E,), jnp.float32),
        mesh=mesh,
        compiler_params=pltpu.CompilerParams(use_tc_tiling_on_sc=False),
        scratch_types=[
            pltpu.VMEM((CHUNK_I,), jnp.int32),
            pltpu.VMEM((CHUNK_I,), jnp.int32),
            pltpu.VMEM((CHUNK_I, E), jnp.float32),
            pltpu.VMEM((CHUNK_I, E), jnp.float32),
            pltpu.VMEM((CB * E,), jnp.float32),
            pltpu.SemaphoreType.DMA,
            pltpu.SemaphoreType.DMA,
        ],
    )
    def body(x_hbm, table_hbm, out_hbm, idx0, idx1, rows0, rows1, stage,
             sem0, sem1):
        wid = lax.axis_index("s") * NC + lax.axis_index("c")
        ibase = wid * (ROWS_PER_W * H)
        obase = wid * (ROWS_PER_W * E)

        idx = (idx0, idx1)
        rows = (rows0, rows1)
        sems = (sem0, sem1)
        handles = [None, None]

        pltpu.sync_copy(x_hbm.at[pl.ds(ibase, CHUNK_I)], idx[0])
        handles[0] = pltpu.async_copy(table_hbm.at[idx[0]], rows[0], sems[0])

        for c in range(NCHUNK):
            cur = c % 2
            nxt = (c + 1) % 2
            if c + 1 < NCHUNK:
                pltpu.sync_copy(
                    x_hbm.at[pl.ds(ibase + (c + 1) * CHUNK_I, CHUNK_I)],
                    idx[nxt])
                handles[nxt] = pltpu.async_copy(
                    table_hbm.at[idx[nxt]], rows[nxt], sems[nxt])
            handles[cur].wait()
            rref = rows[cur]

            def row_body(bi, _, rref=rref):
                base = bi * H
                a0 = rref[base, 0:16]
                a1 = rref[base, 16:32]
                for j in range(1, H):
                    a0 = a0 + rref[base + j, 0:16]
                    a1 = a1 + rref[base + j, 16:32]
                stage[pl.ds(bi * E, 16)] = a0
                stage[pl.ds(bi * E + 16, 16)] = a1
                return 0

            lax.fori_loop(0, CB, row_body, 0)
            pltpu.sync_copy(
                stage, out_hbm.at[pl.ds(obase + c * (CB * E), CB * E)])

    return body(xflat, table)


def _tc_tail(pooled_sum, W, b2):
    BB = 2048

    def body(p_ref, w_ref, b_ref, o_ref):
        p = p_ref[...] * (1.0 / H)
        h = jnp.dot(p, w_ref[...], preferred_element_type=jnp.float32)
        h = jnp.maximum(h + b_ref[...], 0.0)
        m = jnp.max(h, axis=1, keepdims=True)
        e = jnp.exp(h - m)
        o_ref[...] = (h - m) - jnp.log(jnp.sum(e, axis=1, keepdims=True))

    return pl.pallas_call(
        body,
        grid=(B // BB,),
        in_specs=[
            pl.BlockSpec((BB, E), lambda i: (i, 0)),
            pl.BlockSpec((E, 2), lambda i: (0, 0)),
            pl.BlockSpec((1, 2), lambda i: (0, 0)),
        ],
        out_specs=pl.BlockSpec((BB, 2), lambda i: (i, 0)),
        out_shape=jax.ShapeDtypeStruct((B, 2), jnp.float32),
    )(pooled_sum, W, b2)


def kernel(x, table, W, b):
    xflat = x.astype(jnp.int32).reshape(B * H)
    u = xflat & 2047
    xperm = (xflat & ~2047) | ((u & 511) << 2) | (u >> 9)
    packed = _tc_repack(table.T)
    vp = packed.shape[0] * packed.shape[1] // E
    table_lin = packed.reshape(vp, E)
    pooled_sum = _sc_pooled_sum(xperm, table_lin).reshape(B, E)
    return _tc_tail(pooled_sum, W, b.reshape(1, 2))


# speedup vs baseline: 2.1057x; 1.2100x over previous
import functools

import jax
import jax.numpy as jnp
from jax import lax
from jax.experimental import pallas as pl
from jax.experimental.pallas import tpu as pltpu
from jax.experimental.pallas import tpu_sc as plsc

B = 16384
H = 50
E = 32
V = 1000000

NC = 2
NS = 16
NW = NC * NS
ROWS_PER_W = B // NW
CB = 32
NCHUNK = ROWS_PER_W // CB
CHUNK_I = CB * H

CW = 512
NFULL = V // CW
TAIL = V - NFULL * CW
MAXI = (NFULL + 1 + NW - 1) // NW

REPACK_BLK = 2048


def _tc_repack(tT):
    BLK = REPACK_BLK
    R = E * 4
    OB = BLK // 4

    def body(t_ref, o_ref):
        v = t_ref[...]
        w = jnp.concatenate(
            [v[:, j * OB:(j + 1) * OB] for j in range(4)], axis=0)
        o_ref[...] = jnp.transpose(w)

    ngrid = pl.cdiv(V, BLK)
    return pl.pallas_call(
        body,
        grid=(ngrid,),
        in_specs=[pl.BlockSpec((E, BLK), lambda i: (0, i))],
        out_specs=pl.BlockSpec((OB, R), lambda i: (i, 0)),
        out_shape=jax.ShapeDtypeStruct((ngrid * OB, R), jnp.float32),
    )(tT)


def _sc_pooled_sum(xflat, table):
    mesh = plsc.VectorSubcoreMesh(core_axis_name="c", subcore_axis_name="s")

    @functools.partial(
        pl.kernel,
        out_type=jax.ShapeDtypeStruct((B * E,), jnp.float32),
        mesh=mesh,
        compiler_params=pltpu.CompilerParams(use_tc_tiling_on_sc=False),
        scratch_types=[
            pltpu.VMEM((CHUNK_I,), jnp.int32),
            pltpu.VMEM((CHUNK_I,), jnp.int32),
            pltpu.VMEM((CHUNK_I, E), jnp.float32),
            pltpu.VMEM((CHUNK_I, E), jnp.float32),
            pltpu.VMEM((CB * E,), jnp.float32),
            pltpu.SemaphoreType.DMA,
            pltpu.SemaphoreType.DMA,
        ],
    )
    def body(x_hbm, table_hbm, out_hbm, idx0, idx1, rows0, rows1, stage,
             sem0, sem1):
        wid = lax.axis_index("s") * NC + lax.axis_index("c")
        ibase = wid * (ROWS_PER_W * H)
        obase = wid * (ROWS_PER_W * E)

        idx = (idx0, idx1)
        rows = (rows0, rows1)
        sems = (sem0, sem1)
        handles = [None, None]

        pltpu.sync_copy(x_hbm.at[pl.ds(ibase, CHUNK_I)], idx[0])
        handles[0] = pltpu.async_copy(table_hbm.at[idx[0]], rows[0], sems[0])

        for c in range(NCHUNK):
            cur = c % 2
            nxt = (c + 1) % 2
            if c + 1 < NCHUNK:
                pltpu.sync_copy(
                    x_hbm.at[pl.ds(ibase + (c + 1) * CHUNK_I, CHUNK_I)],
                    idx[nxt])
                handles[nxt] = pltpu.async_copy(
                    table_hbm.at[idx[nxt]], rows[nxt], sems[nxt])
            handles[cur].wait()
            rref = rows[cur]

            def row_body(bi, _, rref=rref):
                base = bi * H
                a0 = rref[base, 0:16]
                a1 = rref[base, 16:32]
                for j in range(1, H):
                    a0 = a0 + rref[base + j, 0:16]
                    a1 = a1 + rref[base + j, 16:32]
                stage[pl.ds(bi * E, 16)] = a0
                stage[pl.ds(bi * E + 16, 16)] = a1
                return 0

            lax.fori_loop(0, CB, row_body, 0)
            pltpu.sync_copy(
                stage, out_hbm.at[pl.ds(obase + c * (CB * E), CB * E)])

    return body(xflat, table)


def _tc_tail(pooled_sum, W, b2):
    BB = 2048

    def body(p_ref, w_ref, b_ref, o_ref):
        p = p_ref[...] * (1.0 / H)
        h = jnp.dot(p, w_ref[...], preferred_element_type=jnp.float32)
        h = jnp.maximum(h + b_ref[...], 0.0)
        m = jnp.max(h, axis=1, keepdims=True)
        e = jnp.exp(h - m)
        o_ref[...] = (h - m) - jnp.log(jnp.sum(e, axis=1, keepdims=True))

    return pl.pallas_call(
        body,
        grid=(B // BB,),
        in_specs=[
            pl.BlockSpec((BB, E), lambda i: (i, 0)),
            pl.BlockSpec((E, 2), lambda i: (0, 0)),
            pl.BlockSpec((1, 2), lambda i: (0, 0)),
        ],
        out_specs=pl.BlockSpec((BB, 2), lambda i: (i, 0)),
        out_shape=jax.ShapeDtypeStruct((B, 2), jnp.float32),
    )(pooled_sum, W, b2)


def kernel(x, table, W, b):
    xflat = x.astype(jnp.int32).reshape(B * H)
    blk = REPACK_BLK
    ob = blk // 4
    u = xflat & (blk - 1)
    xperm = (xflat & ~(blk - 1)) | ((u & (ob - 1)) << 2) | (u >> (ob.bit_length() - 1))
    packed = _tc_repack(table.T)
    vp = packed.shape[0] * packed.shape[1] // E
    table_lin = packed.reshape(vp, E)
    pooled_sum = _sc_pooled_sum(xperm, table_lin).reshape(B, E)
    return _tc_tail(pooled_sum, W, b.reshape(1, 2))


# speedup vs baseline: 3.6605x; 1.7384x over previous
import functools

import jax
import jax.numpy as jnp
from jax import lax
from jax.experimental import pallas as pl
from jax.experimental.pallas import tpu as pltpu
from jax.experimental.pallas import tpu_sc as plsc

B = 16384
H = 50
E = 32
V = 1000000

NC = 2
NS = 16
NW = NC * NS
ROWS_PER_W = B // NW
CB = 32
NCHUNK = ROWS_PER_W // CB
CHUNK_I = CB * H

CW = 512
NFULL = V // CW
TAIL = V - NFULL * CW
MAXI = (NFULL + 1 + NW - 1) // NW

REPACK_BLK = 8192


def _tc_repack(tT):
    BLK = REPACK_BLK
    R = E * 4
    OB = BLK // 4

    def body(t_ref, o_ref):
        v = t_ref[...]
        w = jnp.concatenate(
            [v[:, j * OB:(j + 1) * OB] for j in range(4)], axis=0)
        o_ref[...] = jnp.transpose(w)

    ngrid = pl.cdiv(V, BLK)
    return pl.pallas_call(
        body,
        grid=(ngrid,),
        in_specs=[pl.BlockSpec((E, BLK), lambda i: (0, i))],
        out_specs=pl.BlockSpec((OB, R), lambda i: (i, 0)),
        out_shape=jax.ShapeDtypeStruct((ngrid * OB, R), jnp.float32),
    )(tT)


def _sc_pooled_sum(xflat, table):
    mesh = plsc.VectorSubcoreMesh(core_axis_name="c", subcore_axis_name="s")

    @functools.partial(
        pl.kernel,
        out_type=jax.ShapeDtypeStruct((B * E,), jnp.float32),
        mesh=mesh,
        compiler_params=pltpu.CompilerParams(use_tc_tiling_on_sc=False),
        scratch_types=[
            pltpu.VMEM((CHUNK_I,), jnp.int32),
            pltpu.VMEM((CHUNK_I,), jnp.int32),
            pltpu.VMEM((CHUNK_I, E), jnp.float32),
            pltpu.VMEM((CHUNK_I, E), jnp.float32),
            pltpu.VMEM((CB * E,), jnp.float32),
            pltpu.SemaphoreType.DMA,
            pltpu.SemaphoreType.DMA,
        ],
    )
    def body(x_hbm, table_hbm, out_hbm, idx0, idx1, rows0, rows1, stage,
             sem0, sem1):
        wid = lax.axis_index("s") * NC + lax.axis_index("c")
        ibase = wid * (ROWS_PER_W * H)
        obase = wid * (ROWS_PER_W * E)

        idx = (idx0, idx1)
        rows = (rows0, rows1)
        sems = (sem0, sem1)
        handles = [None, None]

        pltpu.sync_copy(x_hbm.at[pl.ds(ibase, CHUNK_I)], idx[0])
        handles[0] = pltpu.async_copy(table_hbm.at[idx[0]], rows[0], sems[0])

        for c in range(NCHUNK):
            cur = c % 2
            nxt = (c + 1) % 2
            if c + 1 < NCHUNK:
                pltpu.sync_copy(
                    x_hbm.at[pl.ds(ibase + (c + 1) * CHUNK_I, CHUNK_I)],
                    idx[nxt])
                handles[nxt] = pltpu.async_copy(
                    table_hbm.at[idx[nxt]], rows[nxt], sems[nxt])
            handles[cur].wait()
            rref = rows[cur]

            def row_body(bi, _, rref=rref):
                base = bi * H
                a0 = rref[base, 0:16]
                a1 = rref[base, 16:32]
                for j in range(1, H):
                    a0 = a0 + rref[base + j, 0:16]
                    a1 = a1 + rref[base + j, 16:32]
                stage[pl.ds(bi * E, 16)] = a0
                stage[pl.ds(bi * E + 16, 16)] = a1
                return 0

            lax.fori_loop(0, CB, row_body, 0)
            pltpu.sync_copy(
                stage, out_hbm.at[pl.ds(obase + c * (CB * E), CB * E)])

    return body(xflat, table)


def _tc_tail(pooled_sum, W, b2):
    BB = 2048

    def body(p_ref, w_ref, b_ref, o_ref):
        p = p_ref[...] * (1.0 / H)
        h = jnp.dot(p, w_ref[...], preferred_element_type=jnp.float32)
        h = jnp.maximum(h + b_ref[...], 0.0)
        m = jnp.max(h, axis=1, keepdims=True)
        e = jnp.exp(h - m)
        o_ref[...] = (h - m) - jnp.log(jnp.sum(e, axis=1, keepdims=True))

    return pl.pallas_call(
        body,
        grid=(B // BB,),
        in_specs=[
            pl.BlockSpec((BB, E), lambda i: (i, 0)),
            pl.BlockSpec((E, 2), lambda i: (0, 0)),
            pl.BlockSpec((1, 2), lambda i: (0, 0)),
        ],
        out_specs=pl.BlockSpec((BB, 2), lambda i: (i, 0)),
        out_shape=jax.ShapeDtypeStruct((B, 2), jnp.float32),
    )(pooled_sum, W, b2)


def kernel(x, table, W, b):
    xflat = x.astype(jnp.int32).reshape(B * H)
    blk = REPACK_BLK
    ob = blk // 4
    u = xflat & (blk - 1)
    xperm = (xflat & ~(blk - 1)) | ((u & (ob - 1)) << 2) | (u >> (ob.bit_length() - 1))
    packed = _tc_repack(table.T)
    vp = packed.shape[0] * packed.shape[1] // E
    table_lin = packed.reshape(vp, E)
    pooled_sum = _sc_pooled_sum(xperm, table_lin).reshape(B, E)
    return _tc_tail(pooled_sum, W, b.reshape(1, 2))


# speedup vs baseline: 4.2977x; 1.1741x over previous
import functools

import jax
import jax.numpy as jnp
from jax import lax
from jax.experimental import pallas as pl
from jax.experimental.pallas import tpu as pltpu
from jax.experimental.pallas import tpu_sc as plsc

B = 16384
H = 50
E = 32
V = 1000000

NC = 2
NS = 16
NW = NC * NS
ROWS_PER_W = B // NW
CB = 32
NCHUNK = ROWS_PER_W // CB
CHUNK_I = CB * H

CW = 512
NFULL = V // CW
TAIL = V - NFULL * CW
MAXI = (NFULL + 1 + NW - 1) // NW

REPACK_BLK = 16384


def _tc_repack(tT):
    BLK = REPACK_BLK
    R = E * 4
    OB = BLK // 4

    def body(t_ref, o_ref):
        v = t_ref[...]
        w = jnp.concatenate(
            [v[:, j * OB:(j + 1) * OB] for j in range(4)], axis=0)
        o_ref[...] = jnp.transpose(w)

    ngrid = pl.cdiv(V, BLK)
    return pl.pallas_call(
        body,
        grid=(ngrid,),
        in_specs=[pl.BlockSpec((E, BLK), lambda i: (0, i))],
        out_specs=pl.BlockSpec((OB, R), lambda i: (i, 0)),
        out_shape=jax.ShapeDtypeStruct((ngrid * OB, R), jnp.float32),
    )(tT)


def _sc_pooled_sum(xflat, table):
    mesh = plsc.VectorSubcoreMesh(core_axis_name="c", subcore_axis_name="s")

    @functools.partial(
        pl.kernel,
        out_type=jax.ShapeDtypeStruct((B * E,), jnp.float32),
        mesh=mesh,
        compiler_params=pltpu.CompilerParams(use_tc_tiling_on_sc=False),
        scratch_types=[
            pltpu.VMEM((CHUNK_I,), jnp.int32),
            pltpu.VMEM((CHUNK_I,), jnp.int32),
            pltpu.VMEM((CHUNK_I, E), jnp.float32),
            pltpu.VMEM((CHUNK_I, E), jnp.float32),
            pltpu.VMEM((CB * E,), jnp.float32),
            pltpu.SemaphoreType.DMA,
            pltpu.SemaphoreType.DMA,
        ],
    )
    def body(x_hbm, table_hbm, out_hbm, idx0, idx1, rows0, rows1, stage,
             sem0, sem1):
        wid = lax.axis_index("s") * NC + lax.axis_index("c")
        ibase = wid * (ROWS_PER_W * H)
        obase = wid * (ROWS_PER_W * E)

        idx = (idx0, idx1)
        rows = (rows0, rows1)
        sems = (sem0, sem1)
        handles = [None, None]

        pltpu.sync_copy(x_hbm.at[pl.ds(ibase, CHUNK_I)], idx[0])
        handles[0] = pltpu.async_copy(table_hbm.at[idx[0]], rows[0], sems[0])

        for c in range(NCHUNK):
            cur = c % 2
            nxt = (c + 1) % 2
            if c + 1 < NCHUNK:
                pltpu.sync_copy(
                    x_hbm.at[pl.ds(ibase + (c + 1) * CHUNK_I, CHUNK_I)],
                    idx[nxt])
                handles[nxt] = pltpu.async_copy(
                    table_hbm.at[idx[nxt]], rows[nxt], sems[nxt])
            handles[cur].wait()
            rref = rows[cur]

            def row_body(bi, _, rref=rref):
                base = bi * H
                a0 = rref[base, 0:16]
                a1 = rref[base, 16:32]
                for j in range(1, H):
                    a0 = a0 + rref[base + j, 0:16]
                    a1 = a1 + rref[base + j, 16:32]
                stage[pl.ds(bi * E, 16)] = a0
                stage[pl.ds(bi * E + 16, 16)] = a1
                return 0

            lax.fori_loop(0, CB, row_body, 0)
            pltpu.sync_copy(
                stage, out_hbm.at[pl.ds(obase + c * (CB * E), CB * E)])

    return body(xflat, table)


def _tc_tail(pooled_sum, W, b2):
    BB = 2048

    def body(p_ref, w_ref, b_ref, o_ref):
        p = p_ref[...] * (1.0 / H)
        h = jnp.dot(p, w_ref[...], preferred_element_type=jnp.float32)
        h = jnp.maximum(h + b_ref[...], 0.0)
        m = jnp.max(h, axis=1, keepdims=True)
        e = jnp.exp(h - m)
        o_ref[...] = (h - m) - jnp.log(jnp.sum(e, axis=1, keepdims=True))

    return pl.pallas_call(
        body,
        grid=(B // BB,),
        in_specs=[
            pl.BlockSpec((BB, E), lambda i: (i, 0)),
            pl.BlockSpec((E, 2), lambda i: (0, 0)),
            pl.BlockSpec((1, 2), lambda i: (0, 0)),
        ],
        out_specs=pl.BlockSpec((BB, 2), lambda i: (i, 0)),
        out_shape=jax.ShapeDtypeStruct((B, 2), jnp.float32),
    )(pooled_sum, W, b2)


def kernel(x, table, W, b):
    xflat = x.astype(jnp.int32).reshape(B * H)
    blk = REPACK_BLK
    ob = blk // 4
    u = xflat & (blk - 1)
    xperm = (xflat & ~(blk - 1)) | ((u & (ob - 1)) << 2) | (u >> (ob.bit_length() - 1))
    packed = _tc_repack(table.T)
    vp = packed.shape[0] * packed.shape[1] // E
    table_lin = packed.reshape(vp, E)
    pooled_sum = _sc_pooled_sum(xperm, table_lin).reshape(B, E)
    return _tc_tail(pooled_sum, W, b.reshape(1, 2))


# speedup vs baseline: 4.6135x; 1.0735x over previous
import functools

import jax
import jax.numpy as jnp
from jax import lax
from jax.experimental import pallas as pl
from jax.experimental.pallas import tpu as pltpu
from jax.experimental.pallas import tpu_sc as plsc

B = 16384
H = 50
E = 32
V = 1000000

NC = 2
NS = 16
NW = NC * NS
ROWS_PER_W = B // NW
CB = 32
NCHUNK = ROWS_PER_W // CB
CHUNK_I = CB * H

CW = 512
NFULL = V // CW
TAIL = V - NFULL * CW
MAXI = (NFULL + 1 + NW - 1) // NW

REPACK_BLK = 32768


def _tc_repack(tT):
    BLK = REPACK_BLK
    R = E * 4
    OB = BLK // 4

    def body(t_ref, o_ref):
        v = t_ref[...]
        w = jnp.concatenate(
            [v[:, j * OB:(j + 1) * OB] for j in range(4)], axis=0)
        o_ref[...] = jnp.transpose(w)

    ngrid = pl.cdiv(V, BLK)
    return pl.pallas_call(
        body,
        grid=(ngrid,),
        in_specs=[pl.BlockSpec((E, BLK), lambda i: (0, i))],
        out_specs=pl.BlockSpec((OB, R), lambda i: (i, 0)),
        out_shape=jax.ShapeDtypeStruct((ngrid * OB, R), jnp.float32),
    )(tT)


def _sc_pooled_sum(xflat, table):
    mesh = plsc.VectorSubcoreMesh(core_axis_name="c", subcore_axis_name="s")

    @functools.partial(
        pl.kernel,
        out_type=jax.ShapeDtypeStruct((B * E,), jnp.float32),
        mesh=mesh,
        compiler_params=pltpu.CompilerParams(use_tc_tiling_on_sc=False),
        scratch_types=[
            pltpu.VMEM((CHUNK_I,), jnp.int32),
            pltpu.VMEM((CHUNK_I,), jnp.int32),
            pltpu.VMEM((CHUNK_I, E), jnp.float32),
            pltpu.VMEM((CHUNK_I, E), jnp.float32),
            pltpu.VMEM((CB * E,), jnp.float32),
            pltpu.SemaphoreType.DMA,
            pltpu.SemaphoreType.DMA,
        ],
    )
    def body(x_hbm, table_hbm, out_hbm, idx0, idx1, rows0, rows1, stage,
             sem0, sem1):
        wid = lax.axis_index("s") * NC + lax.axis_index("c")
        ibase = wid * (ROWS_PER_W * H)
        obase = wid * (ROWS_PER_W * E)

        idx = (idx0, idx1)
        rows = (rows0, rows1)
        sems = (sem0, sem1)
        handles = [None, None]

        pltpu.sync_copy(x_hbm.at[pl.ds(ibase, CHUNK_I)], idx[0])
        handles[0] = pltpu.async_copy(table_hbm.at[idx[0]], rows[0], sems[0])

        for c in range(NCHUNK):
            cur = c % 2
            nxt = (c + 1) % 2
            if c + 1 < NCHUNK:
                pltpu.sync_copy(
                    x_hbm.at[pl.ds(ibase + (c + 1) * CHUNK_I, CHUNK_I)],
                    idx[nxt])
                handles[nxt] = pltpu.async_copy(
                    table_hbm.at[idx[nxt]], rows[nxt], sems[nxt])
            handles[cur].wait()
            rref = rows[cur]

            def row_body(bi, _, rref=rref):
                base = bi * H
                a0 = rref[base, 0:16]
                a1 = rref[base, 16:32]
                for j in range(1, H):
                    a0 = a0 + rref[base + j, 0:16]
                    a1 = a1 + rref[base + j, 16:32]
                stage[pl.ds(bi * E, 16)] = a0
                stage[pl.ds(bi * E + 16, 16)] = a1
                return 0

            lax.fori_loop(0, CB, row_body, 0)
            pltpu.sync_copy(
                stage, out_hbm.at[pl.ds(obase + c * (CB * E), CB * E)])

    return body(xflat, table)


def _tc_tail(pooled_sum, W, b2):
    BB = 2048

    def body(p_ref, w_ref, b_ref, o_ref):
        p = p_ref[...] * (1.0 / H)
        h = jnp.dot(p, w_ref[...], preferred_element_type=jnp.float32)
        h = jnp.maximum(h + b_ref[...], 0.0)
        m = jnp.max(h, axis=1, keepdims=True)
        e = jnp.exp(h - m)
        o_ref[...] = (h - m) - jnp.log(jnp.sum(e, axis=1, keepdims=True))

    return pl.pallas_call(
        body,
        grid=(B // BB,),
        in_specs=[
            pl.BlockSpec((BB, E), lambda i: (i, 0)),
            pl.BlockSpec((E, 2), lambda i: (0, 0)),
            pl.BlockSpec((1, 2), lambda i: (0, 0)),
        ],
        out_specs=pl.BlockSpec((BB, 2), lambda i: (i, 0)),
        out_shape=jax.ShapeDtypeStruct((B, 2), jnp.float32),
    )(pooled_sum, W, b2)


def kernel(x, table, W, b):
    xflat = x.astype(jnp.int32).reshape(B * H)
    blk = REPACK_BLK
    ob = blk // 4
    u = xflat & (blk - 1)
    xperm = (xflat & ~(blk - 1)) | ((u & (ob - 1)) << 2) | (u >> (ob.bit_length() - 1))
    packed = _tc_repack(table.T)
    vp = packed.shape[0] * packed.shape[1] // E
    table_lin = packed.reshape(vp, E)
    pooled_sum = _sc_pooled_sum(xperm, table_lin).reshape(B, E)
    return _tc_tail(pooled_sum, W, b.reshape(1, 2))


# speedup vs baseline: 4.6445x; 1.0067x over previous
import functools

import jax
import jax.numpy as jnp
from jax import lax
from jax.experimental import pallas as pl
from jax.experimental.pallas import tpu as pltpu
from jax.experimental.pallas import tpu_sc as plsc

B = 16384
H = 50
E = 32
V = 1000000

NC = 2
NS = 16
NW = NC * NS
ROWS_PER_W = B // NW
CB = 32
NCHUNK = ROWS_PER_W // CB
CHUNK_I = CB * H

CW = 512
NFULL = V // CW
TAIL = V - NFULL * CW
MAXI = (NFULL + 1 + NW - 1) // NW

REPACK_BLK = 65536


def _tc_repack(tT):
    BLK = REPACK_BLK
    R = E * 4
    OB = BLK // 4

    def body(t_ref, o_ref):
        v = t_ref[...]
        w = jnp.concatenate(
            [v[:, j * OB:(j + 1) * OB] for j in range(4)], axis=0)
        o_ref[...] = jnp.transpose(w)

    ngrid = pl.cdiv(V, BLK)
    return pl.pallas_call(
        body,
        grid=(ngrid,),
        in_specs=[pl.BlockSpec((E, BLK), lambda i: (0, i))],
        out_specs=pl.BlockSpec((OB, R), lambda i: (i, 0)),
        out_shape=jax.ShapeDtypeStruct((ngrid * OB, R), jnp.float32),
    )(tT)


def _sc_pooled_sum(xflat, table):
    mesh = plsc.VectorSubcoreMesh(core_axis_name="c", subcore_axis_name="s")

    @functools.partial(
        pl.kernel,
        out_type=jax.ShapeDtypeStruct((B * E,), jnp.float32),
        mesh=mesh,
        compiler_params=pltpu.CompilerParams(use_tc_tiling_on_sc=False),
        scratch_types=[
            pltpu.VMEM((CHUNK_I,), jnp.int32),
            pltpu.VMEM((CHUNK_I,), jnp.int32),
            pltpu.VMEM((CHUNK_I, E), jnp.float32),
            pltpu.VMEM((CHUNK_I, E), jnp.float32),
            pltpu.VMEM((CB * E,), jnp.float32),
            pltpu.SemaphoreType.DMA,
            pltpu.SemaphoreType.DMA,
        ],
    )
    def body(x_hbm, table_hbm, out_hbm, idx0, idx1, rows0, rows1, stage,
             sem0, sem1):
        wid = lax.axis_index("s") * NC + lax.axis_index("c")
        ibase = wid * (ROWS_PER_W * H)
        obase = wid * (ROWS_PER_W * E)

        idx = (idx0, idx1)
        rows = (rows0, rows1)
        sems = (sem0, sem1)
        handles = [None, None]

        pltpu.sync_copy(x_hbm.at[pl.ds(ibase, CHUNK_I)], idx[0])
        handles[0] = pltpu.async_copy(table_hbm.at[idx[0]], rows[0], sems[0])

        for c in range(NCHUNK):
            cur = c % 2
            nxt = (c + 1) % 2
            if c + 1 < NCHUNK:
                pltpu.sync_copy(
                    x_hbm.at[pl.ds(ibase + (c + 1) * CHUNK_I, CHUNK_I)],
                    idx[nxt])
                handles[nxt] = pltpu.async_copy(
                    table_hbm.at[idx[nxt]], rows[nxt], sems[nxt])
            handles[cur].wait()
            rref = rows[cur]

            def row_body(bi, _, rref=rref):
                base = bi * H
                a0 = rref[base, 0:16]
                a1 = rref[base, 16:32]
                for j in range(1, H):
                    a0 = a0 + rref[base + j, 0:16]
                    a1 = a1 + rref[base + j, 16:32]
                stage[pl.ds(bi * E, 16)] = a0
                stage[pl.ds(bi * E + 16, 16)] = a1
                return 0

            lax.fori_loop(0, CB, row_body, 0)
            pltpu.sync_copy(
                stage, out_hbm.at[pl.ds(obase + c * (CB * E), CB * E)])

    return body(xflat, table)


def _tc_tail(pooled_sum, W, b2):
    BB = 2048

    def body(p_ref, w_ref, b_ref, o_ref):
        p = p_ref[...] * (1.0 / H)
        h = jnp.dot(p, w_ref[...], preferred_element_type=jnp.float32)
        h = jnp.maximum(h + b_ref[...], 0.0)
        m = jnp.max(h, axis=1, keepdims=True)
        e = jnp.exp(h - m)
        o_ref[...] = (h - m) - jnp.log(jnp.sum(e, axis=1, keepdims=True))

    return pl.pallas_call(
        body,
        grid=(B // BB,),
        in_specs=[
            pl.BlockSpec((BB, E), lambda i: (i, 0)),
            pl.BlockSpec((E, 2), lambda i: (0, 0)),
            pl.BlockSpec((1, 2), lambda i: (0, 0)),
        ],
        out_specs=pl.BlockSpec((BB, 2), lambda i: (i, 0)),
        out_shape=jax.ShapeDtypeStruct((B, 2), jnp.float32),
    )(pooled_sum, W, b2)


def kernel(x, table, W, b):
    xflat = x.astype(jnp.int32).reshape(B * H)
    blk = REPACK_BLK
    ob = blk // 4
    u = xflat & (blk - 1)
    xperm = (xflat & ~(blk - 1)) | ((u & (ob - 1)) << 2) | (u >> (ob.bit_length() - 1))
    packed = _tc_repack(table.T)
    vp = packed.shape[0] * packed.shape[1] // E
    table_lin = packed.reshape(vp, E)
    pooled_sum = _sc_pooled_sum(xperm, table_lin).reshape(B, E)
    return _tc_tail(pooled_sum, W, b.reshape(1, 2))
